# Initial kernel scaffold; baseline (speedup 1.0000x reference)
import jax
import jax.numpy as jnp
from jax.experimental import pallas as pl


def kernel(inputs, tok_emb, pos_emb, gamma, beta):
    raise NotImplementedError("write your pallas kernel here")


# speedup vs baseline: 1.1956x; 1.1956x over previous
import functools

import jax
import jax.numpy as jnp
from jax import lax
from jax.experimental import pallas as pl
from jax.experimental.pallas import tpu as pltpu
from jax.experimental.pallas import tpu_sc as plsc

VOCAB = 100000
MAXLEN = 2048
HIDDEN = 768
BATCH = 4
SEQ = 2048

NTOK = BATCH * SEQ
NW = 32
TOK_PER_W = NTOK // NW
CHUNK = 128
NCHUNK = TOK_PER_W // CHUNK


def _sc_gather(idx_flat, tok_emb):
    mesh = plsc.VectorSubcoreMesh(core_axis_name="c", subcore_axis_name="s")

    @functools.partial(
        pl.kernel,
        out_type=jax.ShapeDtypeStruct((NTOK, HIDDEN), jnp.float32),
        mesh=mesh,
        scratch_types=[
            pltpu.VMEM((CHUNK,), jnp.int32),
            pltpu.VMEM((CHUNK, HIDDEN), jnp.float32),
            pltpu.SemaphoreType.DMA,
        ],
    )
    def k(idx_hbm, table_hbm, out_hbm, idx_v, rows_v, sem):
        wid = lax.axis_index("s") * 2 + lax.axis_index("c")
        base = wid * TOK_PER_W
        for i in range(NCHUNK):
            off = base + i * CHUNK
            pltpu.sync_copy(idx_hbm.at[pl.ds(off, CHUNK)], idx_v)
            pltpu.async_copy(table_hbm.at[idx_v], rows_v, sem).wait()
            pltpu.sync_copy(rows_v, out_hbm.at[pl.ds(off, CHUNK)])

    return k(idx_flat, tok_emb)


S_BLK = 512
NS_BLK = SEQ // S_BLK


def _ln_body(x_ref, pos_ref, gb_ref, out_ref):
    x = x_ref[0] + pos_ref[...]
    mean = jnp.mean(x, axis=-1, keepdims=True)
    cent = x - mean
    var = jnp.mean(cent * cent, axis=-1, keepdims=True)
    normed = cent * lax.rsqrt(var + 1e-12)
    out_ref[0] = normed * gb_ref[0, 0][None] + gb_ref[1, 0][None]


def _tc_layernorm(gathered, pos_emb, gamma, beta):
    gb = jnp.stack([gamma, beta]).reshape(2, 1, HIDDEN)
    grid = (NS_BLK, BATCH)
    return pl.pallas_call(
        _ln_body,
        grid=grid,
        in_specs=[
            pl.BlockSpec((1, S_BLK, HIDDEN), lambda s, b: (b, s, 0)),
            pl.BlockSpec((S_BLK, HIDDEN), lambda s, b: (s, 0)),
            pl.BlockSpec((2, 1, HIDDEN), lambda s, b: (0, 0, 0)),
        ],
        out_specs=pl.BlockSpec((1, S_BLK, HIDDEN), lambda s, b: (b, s, 0)),
        out_shape=jax.ShapeDtypeStruct((BATCH, SEQ, HIDDEN), jnp.float32),
    )(gathered, pos_emb, gb)


def kernel(inputs, tok_emb, pos_emb, gamma, beta):
    idx_flat = inputs.reshape(NTOK).astype(jnp.int32)
    gathered = _sc_gather(idx_flat, tok_emb)
    gathered = gathered.reshape(BATCH, SEQ, HIDDEN)
    return _tc_layernorm(gathered, pos_emb, gamma, beta)
